# baseline (speedup 1.0000x reference)
import functools

import jax
import jax.numpy as jnp
from jax import lax
from jax.experimental import pallas as pl
from jax.experimental.pallas import tpu as pltpu
from jax.experimental.pallas import tpu_sc as plsc

B = 16384
D = 50
DP = 128
DU = 64
HIDDEN = 64
NC = 2
NS = 16
NW = NC * NS
BPW = B // NW
NCHUNK = 4
CHUNK = BPW // NCHUNK
KRING = 8
LANES = 16


def _sc_rowdma3(t0, t1, t2, i0, i1, i2):
    mesh = plsc.VectorSubcoreMesh(core_axis_name="c", subcore_axis_name="s")
    out_type = tuple(jax.ShapeDtypeStruct((B, D), jnp.float32) for _ in range(3))
    scratch = [
        pltpu.VMEM((BPW,), jnp.int32),
        pltpu.VMEM((BPW,), jnp.int32),
        pltpu.VMEM((BPW,), jnp.int32),
        pltpu.VMEM((BPW, D), jnp.float32),
        pltpu.SemaphoreType.DMA,
    ]

    @functools.partial(pl.kernel, mesh=mesh, out_type=out_type,
                       scratch_types=scratch,
                       compiler_params=pltpu.CompilerParams(
                           use_tc_tiling_on_sc=True))
    def k(t0, t1, t2, i0, i1, i2, o0, o1, o2, vi0, vi1, vi2, vr, sem):
        wid = lax.axis_index("s") * NC + lax.axis_index("c")
        base = wid * BPW
        tabs = (t0, t1, t2)
        idxs = (i0, i1, i2)
        outs = (o0, o1, o2)
        vis = (vi0, vi1, vi2)
        for t in range(3):
            pltpu.sync_copy(idxs[t].at[wid], vis[t])
        for t in range(3):
            tab, vi = tabs[t], vis[t]

            def issue(j, _, tab=tab, vi=vi):
                iv = vi[pl.ds(j * LANES, LANES)]
                for l in range(LANES):
                    r = j * LANES + l
                    pltpu.async_copy(
                        tab.at[pl.ds(iv[l], 1), :],
                        vr.at[pl.ds(r, 1), :],
                        sem)
                return 0

            lax.fori_loop(0, BPW // LANES, issue, 0, unroll=False)
            pltpu.make_async_copy(tab.at[pl.ds(0, BPW), :], vr, sem).wait()
            pltpu.sync_copy(vr, outs[t].at[pl.ds(base, BPW)])

    return k(t0, t1, t2, i0, i1, i2)


def _sc_user_gather(tabT, idx):
    mesh = plsc.VectorSubcoreMesh(core_axis_name="c", subcore_axis_name="s")
    scratch = [
        pltpu.VMEM((BPW,), jnp.int32),
        pltpu.VMEM((BPW, DU), jnp.float32),
    ]
    for _ in range(KRING):
        scratch.append(pltpu.VMEM((D, 128), jnp.float32))
    for _ in range(KRING):
        scratch.append(pltpu.SemaphoreType.DMA)

    @functools.partial(pl.kernel, mesh=mesh,
                       out_type=jax.ShapeDtypeStruct((B, DU), jnp.float32),
                       scratch_types=scratch,
                       compiler_params=pltpu.CompilerParams(
                           use_tc_tiling_on_sc=True,
                           needs_layout_passes=False))
    def k(tabT, idx, out, vi, stage, *ring_and_sems):
        ring = ring_and_sems[:KRING]
        sems = ring_and_sems[KRING:]
        wid = lax.axis_index("s") * NC + lax.axis_index("c")
        base = wid * BPW
        pltpu.sync_copy(idx.at[wid], vi)

        def fetch(v, slot):
            off = pl.multiple_of((v >> 7) * 128, 128)
            pltpu.async_copy(tabT.at[:, pl.ds(off, 128)], ring[slot],
                             sems[slot])

        def extract(v, r, slot):
            c = jnp.broadcast_to(v & 127, (LANES,))
            for a in range(4):
                rows = lax.iota(jnp.int32, LANES) + a * LANES
                if a < 3:
                    col = plsc.load_gather(ring[slot], [rows, c])
                else:
                    mask = lax.iota(jnp.int32, LANES) < (D - 3 * LANES)
                    col = plsc.load_gather(ring[slot], [rows, c], mask=mask)
                    col = jnp.where(mask, col, 0.0)
                stage[r, pl.ds(a * LANES, LANES)] = col

        iv0 = vi[pl.ds(0, LANES)]
        for s in range(KRING):
            fetch(iv0[s], s)

        ngroups = BPW // LANES

        def body(q, _):
            iv = vi[pl.ds(q * LANES, LANES)]
            qn = jnp.minimum(q + 1, ngroups - 1)
            ivn = vi[pl.ds(qn * LANES, LANES)]
            for s in range(LANES):
                r = q * LANES + s
                slot = s % KRING
                pltpu.make_async_copy(
                    tabT.at[:, pl.ds(0, 128)], ring[slot], sems[slot]).wait()
                extract(iv[s], r, slot)
                vnext = iv[s + KRING] if s + KRING < LANES \
                    else ivn[s + KRING - LANES]

                @pl.when(r + KRING < BPW)
                def _(vnext=vnext, slot=slot):
                    fetch(vnext, slot)
            return 0

        lax.fori_loop(0, ngroups, body, 0, unroll=False)
        pltpu.sync_copy(stage, out.at[pl.ds(base, BPW)])

    return k(tabT, idx)


def _tc_pad(tab):
    V = tab.shape[0]
    bm = min(V, 4096)

    def body(x_ref, o_ref):
        o_ref[...] = jnp.concatenate(
            [x_ref[...], jnp.zeros((bm, DP - D), jnp.float32)], axis=1)

    return pl.pallas_call(
        body,
        grid=(V // bm,),
        in_specs=[pl.BlockSpec((bm, D), lambda i: (i, 0))],
        out_specs=pl.BlockSpec((bm, DP), lambda i: (i, 0)),
        out_shape=jax.ShapeDtypeStruct((V, DP), jnp.float32),
    )(tab)


def _tc_mlp(xc, xi, xr, xu, w1c, w1i, w1r, w1u, b1, wo, bo, offset):
    bm = 2048
    grid = (B // bm,)

    def body(xc_ref, xi_ref, xr_ref, xu_ref, w1c_ref, w1i_ref, w1r_ref,
             w1u_ref, b1_ref, wo_ref, bo_ref, off_ref, out_ref):
        h = jnp.dot(xc_ref[...], w1c_ref[...], preferred_element_type=jnp.float32)
        h = h + jnp.dot(xi_ref[...], w1i_ref[...], preferred_element_type=jnp.float32)
        h = h + jnp.dot(xr_ref[...], w1r_ref[...], preferred_element_type=jnp.float32)
        h = h + jnp.dot(xu_ref[...], w1u_ref[...], preferred_element_type=jnp.float32)
        h = jnp.maximum(h + b1_ref[...], 0.0)
        y = jnp.sum(h * wo_ref[...], axis=1) + bo_ref[0, 0]
        out_ref[...] = y + off_ref[...]

    xp_spec = pl.BlockSpec((bm, D), lambda i: (i, 0))
    wp_spec = pl.BlockSpec((D, HIDDEN), lambda i: (0, 0))
    return pl.pallas_call(
        body,
        grid=grid,
        in_specs=[
            xp_spec, xp_spec, xp_spec,
            pl.BlockSpec((bm, DU), lambda i: (i, 0)),
            wp_spec, wp_spec, wp_spec,
            pl.BlockSpec((DU, HIDDEN), lambda i: (0, 0)),
            pl.BlockSpec((1, HIDDEN), lambda i: (0, 0)),
            pl.BlockSpec((1, HIDDEN), lambda i: (0, 0)),
            pl.BlockSpec((1, 1), lambda i: (0, 0)),
            pl.BlockSpec((bm,), lambda i: (i,)),
        ],
        out_specs=pl.BlockSpec((bm,), lambda i: (i,)),
        out_shape=jax.ShapeDtypeStruct((B,), jnp.float32),
    )(xc, xi, xr, xu, w1c, w1i, w1r, w1u, b1, wo, bo, offset)


def kernel(user_id, item_id, category_id, region_id, offset,
           E_category, E_item, E_region, E_user, W1, b1, W_out, b_out):
    ic = category_id.astype(jnp.int32).reshape(NW, BPW)
    ii = item_id.astype(jnp.int32).reshape(NW, BPW)
    ir = region_id.astype(jnp.int32).reshape(NW, BPW)
    iu = user_id.astype(jnp.int32).reshape(NW, BPW)
    xc, xi, xr = _sc_rowdma3(E_category, E_item, E_region, ic, ii, ir)
    xu = _sc_user_gather(jnp.transpose(E_user), iu)
    w1c = W1[0:D]
    w1i = W1[D:2 * D]
    w1r = W1[2 * D:3 * D]
    w1u = jnp.pad(W1[3 * D:4 * D], ((0, DU - D), (0, 0)))
    return _tc_mlp(xc, xi, xr, xu, w1c, w1i, w1r, w1u,
                   b1.reshape(1, HIDDEN), W_out.reshape(1, HIDDEN),
                   b_out.reshape(1, 1), offset)


# speedup vs baseline: 1.0036x; 1.0036x over previous
import functools

import jax
import jax.numpy as jnp
from jax import lax
from jax.experimental import pallas as pl
from jax.experimental.pallas import tpu as pltpu
from jax.experimental.pallas import tpu_sc as plsc

B = 16384
D = 50
DU = 64
HIDDEN = 64
NC = 2
NS = 16
NW = NC * NS
BPW = B // NW
KRING = 8
LANES = 16


def _sc_rowdma3(t0, t1, t2, i0, i1, i2):
    mesh = plsc.VectorSubcoreMesh(core_axis_name="c", subcore_axis_name="s")
    out_type = tuple(jax.ShapeDtypeStruct((B, D), jnp.float32) for _ in range(3))
    scratch = [
        pltpu.VMEM((BPW,), jnp.int32),
        pltpu.VMEM((BPW,), jnp.int32),
        pltpu.VMEM((BPW,), jnp.int32),
        pltpu.VMEM((BPW, D), jnp.float32),
        pltpu.SemaphoreType.DMA,
    ]

    @functools.partial(pl.kernel, mesh=mesh, out_type=out_type,
                       scratch_types=scratch,
                       compiler_params=pltpu.CompilerParams(
                           use_tc_tiling_on_sc=True))
    def k(t0, t1, t2, i0, i1, i2, o0, o1, o2, vi0, vi1, vi2, vr, sem):
        wid = lax.axis_index("s") * NC + lax.axis_index("c")
        base = wid * BPW
        tabs = (t0, t1, t2)
        idxs = (i0, i1, i2)
        outs = (o0, o1, o2)
        vis = (vi0, vi1, vi2)
        for t in range(3):
            pltpu.sync_copy(idxs[t].at[wid], vis[t])
        for t in range(3):
            tab, vi = tabs[t], vis[t]

            def issue(j, _, tab=tab, vi=vi):
                iv = vi[pl.ds(j * LANES, LANES)]
                for l in range(LANES):
                    r = j * LANES + l
                    pltpu.async_copy(
                        tab.at[pl.ds(iv[l], 1), :],
                        vr.at[pl.ds(r, 1), :],
                        sem)
                return 0

            lax.fori_loop(0, BPW // LANES, issue, 0, unroll=False)
            pltpu.make_async_copy(tab.at[pl.ds(0, BPW), :], vr, sem).wait()
            pltpu.sync_copy(vr, outs[t].at[pl.ds(base, BPW)])

    return k(t0, t1, t2, i0, i1, i2)


def _sc_user_gather(tabT, idx):
    mesh = plsc.VectorSubcoreMesh(core_axis_name="c", subcore_axis_name="s")
    scratch = [
        pltpu.VMEM((BPW,), jnp.int32),
        pltpu.VMEM((BPW, DU), jnp.float32),
    ]
    for _ in range(KRING):
        scratch.append(pltpu.VMEM((D, 128), jnp.float32))
    for _ in range(KRING):
        scratch.append(pltpu.SemaphoreType.DMA)

    @functools.partial(pl.kernel, mesh=mesh,
                       out_type=jax.ShapeDtypeStruct((B, DU), jnp.float32),
                       scratch_types=scratch,
                       compiler_params=pltpu.CompilerParams(
                           use_tc_tiling_on_sc=True,
                           needs_layout_passes=False))
    def k(tabT, idx, out, vi, stage, *ring_and_sems):
        ring = ring_and_sems[:KRING]
        sems = ring_and_sems[KRING:]
        wid = lax.axis_index("s") * NC + lax.axis_index("c")
        base = wid * BPW
        pltpu.sync_copy(idx.at[wid], vi)

        def fetch(v, slot):
            off = pl.multiple_of((v >> 7) * 128, 128)
            pltpu.async_copy(tabT.at[:, pl.ds(off, 128)], ring[slot],
                             sems[slot])

        def extract(v, r, slot):
            c = jnp.broadcast_to(v & 127, (LANES,))
            for a in range(4):
                rows = lax.iota(jnp.int32, LANES) + a * LANES
                if a < 3:
                    col = plsc.load_gather(ring[slot], [rows, c])
                else:
                    mask = lax.iota(jnp.int32, LANES) < (D - 3 * LANES)
                    col = plsc.load_gather(ring[slot], [rows, c], mask=mask)
                    col = jnp.where(mask, col, 0.0)
                stage[r, pl.ds(a * LANES, LANES)] = col

        iv0 = vi[pl.ds(0, LANES)]
        for s in range(KRING):
            fetch(iv0[s], s)

        ngroups = BPW // LANES

        def body(q, _):
            iv = vi[pl.ds(q * LANES, LANES)]
            qn = jnp.minimum(q + 1, ngroups - 1)
            ivn = vi[pl.ds(qn * LANES, LANES)]
            for s in range(LANES):
                r = q * LANES + s
                slot = s % KRING
                pltpu.make_async_copy(
                    tabT.at[:, pl.ds(0, 128)], ring[slot], sems[slot]).wait()
                extract(iv[s], r, slot)
                vnext = iv[s + KRING] if s + KRING < LANES \
                    else ivn[s + KRING - LANES]

                @pl.when(r + KRING < BPW)
                def _(vnext=vnext, slot=slot):
                    fetch(vnext, slot)
            return 0

        lax.fori_loop(0, ngroups, body, 0, unroll=False)
        pltpu.sync_copy(stage, out.at[pl.ds(base, BPW)])

    return k(tabT, idx)


def _tc_mlp(xc, xi, xr, xu, w1c, w1i, w1r, w1u, b1, wo, bo, offset):
    bm = 2048
    grid = (B // bm,)

    def body(xc_ref, xi_ref, xr_ref, xu_ref, w1c_ref, w1i_ref, w1r_ref,
             w1u_ref, b1_ref, wo_ref, bo_ref, off_ref, out_ref):
        h = jnp.dot(xc_ref[...], w1c_ref[...], preferred_element_type=jnp.float32)
        h = h + jnp.dot(xi_ref[...], w1i_ref[...], preferred_element_type=jnp.float32)
        h = h + jnp.dot(xr_ref[...], w1r_ref[...], preferred_element_type=jnp.float32)
        h = h + jnp.dot(xu_ref[...], w1u_ref[...], preferred_element_type=jnp.float32)
        h = jnp.maximum(h + b1_ref[...], 0.0)
        y = jnp.sum(h * wo_ref[...], axis=1) + bo_ref[0, 0]
        out_ref[...] = y + off_ref[...]

    xp_spec = pl.BlockSpec((bm, D), lambda i: (i, 0))
    wp_spec = pl.BlockSpec((D, HIDDEN), lambda i: (0, 0))
    return pl.pallas_call(
        body,
        grid=grid,
        in_specs=[
            xp_spec, xp_spec, xp_spec,
            pl.BlockSpec((bm, DU), lambda i: (i, 0)),
            wp_spec, wp_spec, wp_spec,
            pl.BlockSpec((DU, HIDDEN), lambda i: (0, 0)),
            pl.BlockSpec((1, HIDDEN), lambda i: (0, 0)),
            pl.BlockSpec((1, HIDDEN), lambda i: (0, 0)),
            pl.BlockSpec((1, 1), lambda i: (0, 0)),
            pl.BlockSpec((bm,), lambda i: (i,)),
        ],
        out_specs=pl.BlockSpec((bm,), lambda i: (i,)),
        out_shape=jax.ShapeDtypeStruct((B,), jnp.float32),
    )(xc, xi, xr, xu, w1c, w1i, w1r, w1u, b1, wo, bo, offset)


def kernel(user_id, item_id, category_id, region_id, offset,
           E_category, E_item, E_region, E_user, W1, b1, W_out, b_out):
    ic = category_id.astype(jnp.int32).reshape(NW, BPW)
    ii = item_id.astype(jnp.int32).reshape(NW, BPW)
    ir = region_id.astype(jnp.int32).reshape(NW, BPW)
    iu = user_id.astype(jnp.int32).reshape(NW, BPW)
    xc, xi, xr = _sc_rowdma3(E_category, E_item, E_region, ic, ii, ir)
    xu = _sc_user_gather(jnp.transpose(E_user), iu)
    w1c = W1[0:D]
    w1i = W1[D:2 * D]
    w1r = W1[2 * D:3 * D]
    w1u = jnp.pad(W1[3 * D:4 * D], ((0, DU - D), (0, 0)))
    return _tc_mlp(xc, xi, xr, xu, w1c, w1i, w1r, w1u,
                   b1.reshape(1, HIDDEN), W_out.reshape(1, HIDDEN),
                   b_out.reshape(1, 1), offset)
